# baseline (speedup 1.0000x reference)
import functools

import jax
import jax.numpy as jnp
from jax import lax
from jax.experimental import pallas as pl
from jax.experimental.pallas import tpu as pltpu
from jax.experimental.pallas import tpu_sc as plsc

D = 256
CHUNK = 128
ZCH = 128
LANES = 16
PA = 2688
PB = 3568
MAXWIN = PA // 128 + 2


@functools.partial(jax.jit, static_argnums=(0, 1, 2))
def _build(rows_out, rows_in, nw, h, idx32):

    mesh = plsc.VectorSubcoreMesh(core_axis_name="c", subcore_axis_name="s")
    nc = mesh.num_cores

    @functools.partial(
        pl.kernel,
        out_type=jax.ShapeDtypeStruct((rows_out, D), jnp.float32),
        mesh=mesh,
        scratch_types=[
            pltpu.VMEM((ZCH, D), jnp.float32),
            pltpu.VMEM((2, CHUNK, D), jnp.float32),
            pltpu.VMEM((MAXWIN, CHUNK), jnp.int32),
            pltpu.VMEM((MAXWIN * CHUNK,), jnp.int32),
            pltpu.VMEM((LANES,), jnp.int32),
            pltpu.VMEM((LANES,), jnp.int32),
            pltpu.VMEM((LANES,), jnp.int32),
            pltpu.VMEM((LANES,), jnp.int32),
            pltpu.SemaphoreType.DMA,
            pltpu.SemaphoreType.DMA,
            pltpu.SemaphoreType.DMA,
            pltpu.SemaphoreType.DMA,
        ],
    )
    def unpool(h_hbm, idx_hbm, out_hbm, zeros_v, rows2_v, idx2_v,
               idxf_v, poss_v, pose_v, prbs_v, prbe_v, semz, semi, semh, sems):
        w = lax.axis_index("s") * nc + lax.axis_index("c")

        def zbody(i, carry):
            r = i // (D // LANES)
            c = (i % (D // LANES)) * LANES
            zeros_v[r, pl.ds(c, LANES)] = jnp.zeros((LANES,), jnp.float32)
            return carry

        lax.fori_loop(0, CHUNK * (D // LANES), zbody, 0)

        c_ax = lax.axis_index("c")
        pair = lax.axis_index("s")
        lo = jnp.minimum(pair * (PA + PB) + c_ax * PA, rows_out)
        size = jnp.where(c_ax == 0, PA, PB)
        hi = jnp.minimum(lo + size, rows_out)
        nfull = (hi - lo) // ZCH

        iot = lax.iota(jnp.int32, LANES)
        ls = jnp.int32(0)
        hs = jnp.int32(rows_in)
        le = jnp.int32(0)
        he = jnp.int32(rows_in)
        for _ in range(4):
            st_s = (hs - ls + LANES - 1) // LANES
            st_e = (he - le + LANES - 1) // LANES
            poss_v[pl.ds(0, LANES)] = jnp.minimum(ls + iot * st_s, rows_in - 1)
            pose_v[pl.ds(0, LANES)] = jnp.minimum(le + iot * st_e, rows_in - 1)
            cp1 = pltpu.make_async_copy(idx_hbm.at[poss_v], prbs_v, semi)
            cp2 = pltpu.make_async_copy(idx_hbm.at[pose_v], prbe_v, semh)
            cp1.start()
            cp2.start()
            cp1.wait()
            cp2.wait()
            pvs = prbs_v[pl.ds(0, LANES)]
            pve = prbe_v[pl.ds(0, LANES)]
            cs = jnp.int32(0)
            ce = jnp.int32(0)
            for i in range(LANES):
                cs = cs + jnp.logical_and(
                    pvs[i] < lo, ls + i * st_s < hs
                ).astype(jnp.int32)
                ce = ce + jnp.logical_and(
                    pve[i] < hi, le + i * st_e < he
                ).astype(jnp.int32)
            new_ls = jnp.where(cs >= 1, ls + (cs - 1) * st_s + 1, ls)
            hs = jnp.minimum(ls + cs * st_s, hs)
            ls = new_ls
            new_le = jnp.where(ce >= 1, le + (ce - 1) * st_e + 1, le)
            he = jnp.minimum(le + ce * st_e, he)
            le = new_le
        s = ls
        e = le

        a0 = (s // 8) * 8
        nwin = (e - a0 + CHUNK - 1) // CHUNK

        def astart(j):
            return jnp.minimum(a0 + j * CHUNK, rows_in - CHUNK)

        def iissue(j, carry):
            pltpu.make_async_copy(
                idx_hbm.at[pl.ds(astart(j), CHUNK)], idx2_v.at[j], semi
            ).start()
            pltpu.make_async_copy(
                idx_hbm.at[pl.ds(astart(j), CHUNK)],
                idxf_v.at[pl.ds(j * CHUNK, CHUNK)], semi
            ).start()
            return carry

        lax.fori_loop(0, nwin, iissue, 0)

        @pl.when(nwin >= 1)
        def _():
            pltpu.make_async_copy(
                h_hbm.at[pl.ds(astart(0), CHUNK)], rows2_v.at[0], semh
            ).start()

        def idrain(j, carry):
            pltpu.make_async_copy(
                idx_hbm.at[pl.ds(0, CHUNK)], idx2_v.at[0], semi
            ).wait()
            return carry

        lax.fori_loop(0, 2 * nwin, idrain, 0)

        base = s - a0
        seglen = e - s

        def full_chunk(b):
            def bstep(i, c):
                blo, bhi = c
                mid = (blo + bhi) // 2
                v = idxf_v[pl.ds(base + mid, LANES)]
                lt = v[0] < b
                return (jnp.where(lt, mid + 1, blo), jnp.where(lt, bhi, mid))

            p, _ = lax.fori_loop(0, 12, bstep, (jnp.int32(0), seglen))
            v0 = idxf_v[pl.ds(base + p, LANES)]
            vn = idxf_v[pl.ds(base + p + ZCH - 1, LANES)]
            return jnp.logical_and(
                p + ZCH <= seglen,
                jnp.logical_and(v0[0] == b, vn[0] == b + ZCH - 1),
            )

        def zissue(j, nz):
            skip = full_chunk(lo + j * ZCH)

            @pl.when(jnp.logical_not(skip))
            def _():
                pltpu.make_async_copy(
                    zeros_v, out_hbm.at[pl.ds(lo + j * ZCH, ZCH)], semz
                ).start()

            return nz + 1 - skip.astype(jnp.int32)

        nz = lax.fori_loop(0, nfull, zissue, jnp.int32(0))
        skip_t = full_chunk(hi - ZCH)

        @pl.when(jnp.logical_not(skip_t))
        def _():
            pltpu.make_async_copy(
                zeros_v, out_hbm.at[pl.ds(hi - ZCH, ZCH)], semz
            ).start()

        nz = nz + 1 - skip_t.astype(jnp.int32)

        def zdrain(j, carry):
            pltpu.make_async_copy(
                zeros_v, out_hbm.at[pl.ds(lo, ZCH)], semz
            ).wait()
            return carry

        lax.fori_loop(0, nz, zdrain, 0)

        def scat(j, carry):
            b = j % 2
            pltpu.make_async_copy(
                h_hbm.at[pl.ds(0, CHUNK)], rows2_v.at[0], semh
            ).wait()

            @pl.when(j >= 1)
            def _():
                pltpu.make_async_copy(
                    rows2_v.at[0], out_hbm.at[idx2_v.at[0]], sems
                ).wait()

            @pl.when(j + 1 < nwin)
            def _():
                pltpu.make_async_copy(
                    h_hbm.at[pl.ds(astart(j + 1), CHUNK)], rows2_v.at[1 - b], semh
                ).start()

            pltpu.make_async_copy(
                rows2_v.at[b], out_hbm.at[idx2_v.at[j]], sems
            ).start()
            return carry

        lax.fori_loop(0, nwin, scat, 0)

        @pl.when(nwin >= 1)
        def _():
            pltpu.make_async_copy(
                rows2_v.at[0], out_hbm.at[idx2_v.at[0]], sems
            ).wait()

    return unpool(h, idx32)


def kernel(g, h, idx):
    rows_out = g.shape[0]
    rows_in = h.shape[0]
    info = plsc.get_sparse_core_info()
    nw = info.num_cores * info.num_subcores

    idx32 = idx.astype(jnp.int32)
    return _build(rows_out, rows_in, nw, h, idx32)


# speedup vs baseline: 1.1232x; 1.1232x over previous
import functools

import jax
import jax.numpy as jnp
from jax import lax
from jax.experimental import pallas as pl
from jax.experimental.pallas import tpu as pltpu
from jax.experimental.pallas import tpu_sc as plsc

D = 256
CHUNK = 128
ZCH = 128
LANES = 16
MAXWIN = 26
NCNT = 48


@functools.partial(jax.jit, static_argnums=(0, 1, 2))
def _build(rows_out, rows_in, nw, h, idx32):
    per = (-(-rows_out // nw) + 7) // 8 * 8
    tail_slot = per // CHUNK + 1

    mesh = plsc.VectorSubcoreMesh(core_axis_name="c", subcore_axis_name="s")
    nc = mesh.num_cores

    @functools.partial(
        pl.kernel,
        out_type=jax.ShapeDtypeStruct((rows_out, D), jnp.float32),
        mesh=mesh,
        scratch_types=[
            pltpu.VMEM((ZCH, D), jnp.float32),
            pltpu.VMEM((2, CHUNK, D), jnp.float32),
            pltpu.VMEM((MAXWIN, CHUNK), jnp.int32),
            pltpu.VMEM((MAXWIN * CHUNK,), jnp.int32),
            pltpu.VMEM((LANES,), jnp.int32),
            pltpu.VMEM((LANES,), jnp.int32),
            pltpu.VMEM((LANES,), jnp.int32),
            pltpu.VMEM((LANES,), jnp.int32),
            pltpu.SemaphoreType.DMA,
            pltpu.SemaphoreType.DMA,
            pltpu.SemaphoreType.DMA,
            pltpu.SemaphoreType.DMA,
        ],
    )
    def unpool(h_hbm, idx_hbm, out_hbm, zeros_v, rows2_v, idx2_v,
               idxf_v, poss_v, pose_v, prbs_v, prbe_v, semz, semi, semh, sems):
        w = lax.axis_index("s") * nc + lax.axis_index("c")

        def zbody(i, carry):
            r = i // (D // LANES)
            c = (i % (D // LANES)) * LANES
            zeros_v[r, pl.ds(c, LANES)] = jnp.zeros((LANES,), jnp.float32)
            return carry

        lax.fori_loop(0, CHUNK * (D // LANES), zbody, 0)

        lo = w * per
        hi = jnp.minimum(lo + per, rows_out)
        nfull = (hi - lo) // ZCH

        iot = lax.iota(jnp.int32, LANES)
        ls = jnp.int32(0)
        hs = jnp.int32(rows_in)
        le = jnp.int32(0)
        he = jnp.int32(rows_in)
        for _ in range(4):
            st_s = (hs - ls + LANES - 1) // LANES
            st_e = (he - le + LANES - 1) // LANES
            poss_v[pl.ds(0, LANES)] = jnp.minimum(ls + iot * st_s, rows_in - 1)
            pose_v[pl.ds(0, LANES)] = jnp.minimum(le + iot * st_e, rows_in - 1)
            cp1 = pltpu.make_async_copy(idx_hbm.at[poss_v], prbs_v, semi)
            cp2 = pltpu.make_async_copy(idx_hbm.at[pose_v], prbe_v, semh)
            cp1.start()
            cp2.start()
            cp1.wait()
            cp2.wait()
            pvs = prbs_v[pl.ds(0, LANES)]
            pve = prbe_v[pl.ds(0, LANES)]
            cs = jnp.int32(0)
            ce = jnp.int32(0)
            for i in range(LANES):
                cs = cs + jnp.logical_and(
                    pvs[i] < lo, ls + i * st_s < hs
                ).astype(jnp.int32)
                ce = ce + jnp.logical_and(
                    pve[i] < hi, le + i * st_e < he
                ).astype(jnp.int32)
            new_ls = jnp.where(cs >= 1, ls + (cs - 1) * st_s + 1, ls)
            hs = jnp.minimum(ls + cs * st_s, hs)
            ls = new_ls
            new_le = jnp.where(ce >= 1, le + (ce - 1) * st_e + 1, le)
            he = jnp.minimum(le + ce * st_e, he)
            le = new_le
        s = ls
        e = le

        a0 = (s // 8) * 8
        nwin = (e - a0 + CHUNK - 1) // CHUNK

        def astart(j):
            return jnp.minimum(a0 + j * CHUNK, rows_in - CHUNK)

        def iissue(j, carry):
            pltpu.make_async_copy(
                idx_hbm.at[pl.ds(astart(j), CHUNK)], idx2_v.at[j], semi
            ).start()
            pltpu.make_async_copy(
                idx_hbm.at[pl.ds(astart(j), CHUNK)],
                idxf_v.at[pl.ds(j * CHUNK, CHUNK)], semi
            ).start()
            return carry

        lax.fori_loop(0, nwin, iissue, 0)

        @pl.when(nwin >= 1)
        def _():
            pltpu.make_async_copy(
                h_hbm.at[pl.ds(astart(0), CHUNK)], rows2_v.at[0], semh
            ).start()

        def idrain(j, carry):
            pltpu.make_async_copy(
                idx_hbm.at[pl.ds(0, CHUNK)], idx2_v.at[0], semi
            ).wait()
            return carry

        lax.fori_loop(0, 2 * nwin, idrain, 0)

        base = s - a0
        seglen = e - s

        def full_chunk(b):
            def bstep(i, c):
                blo, bhi = c
                mid = (blo + bhi) // 2
                v = idxf_v[pl.ds(base + mid, LANES)]
                lt = v[0] < b
                return (jnp.where(lt, mid + 1, blo), jnp.where(lt, bhi, mid))

            p, _ = lax.fori_loop(0, 12, bstep, (jnp.int32(0), seglen))
            v0 = idxf_v[pl.ds(base + p, LANES)]
            vn = idxf_v[pl.ds(base + p + ZCH - 1, LANES)]
            return jnp.logical_and(
                p + ZCH <= seglen,
                jnp.logical_and(v0[0] == b, vn[0] == b + ZCH - 1),
            )

        def zissue(j, nz):
            skip = full_chunk(lo + j * ZCH)

            @pl.when(jnp.logical_not(skip))
            def _():
                pltpu.make_async_copy(
                    zeros_v, out_hbm.at[pl.ds(lo + j * ZCH, ZCH)], semz
                ).start()

            return nz + 1 - skip.astype(jnp.int32)

        nz = lax.fori_loop(0, nfull, zissue, jnp.int32(0))
        skip_t = full_chunk(hi - ZCH)

        @pl.when(jnp.logical_not(skip_t))
        def _():
            pltpu.make_async_copy(
                zeros_v, out_hbm.at[pl.ds(hi - ZCH, ZCH)], semz
            ).start()

        nz = nz + 1 - skip_t.astype(jnp.int32)

        def zdrain(j, carry):
            pltpu.make_async_copy(
                zeros_v, out_hbm.at[pl.ds(lo, ZCH)], semz
            ).wait()
            return carry

        lax.fori_loop(0, nz, zdrain, 0)

        def scat(j, carry):
            b = j % 2
            pltpu.make_async_copy(
                h_hbm.at[pl.ds(0, CHUNK)], rows2_v.at[0], semh
            ).wait()

            @pl.when(j >= 1)
            def _():
                pltpu.make_async_copy(
                    rows2_v.at[0], out_hbm.at[idx2_v.at[0]], sems
                ).wait()

            @pl.when(j + 1 < nwin)
            def _():
                pltpu.make_async_copy(
                    h_hbm.at[pl.ds(astart(j + 1), CHUNK)], rows2_v.at[1 - b], semh
                ).start()

            pltpu.make_async_copy(
                rows2_v.at[b], out_hbm.at[idx2_v.at[j]], sems
            ).start()
            return carry

        lax.fori_loop(0, nwin, scat, 0)

        @pl.when(nwin >= 1)
        def _():
            pltpu.make_async_copy(
                rows2_v.at[0], out_hbm.at[idx2_v.at[0]], sems
            ).wait()

    return unpool(h, idx32)


def kernel(g, h, idx):
    rows_out = g.shape[0]
    rows_in = h.shape[0]
    info = plsc.get_sparse_core_info()
    nw = info.num_cores * info.num_subcores

    idx32 = idx.astype(jnp.int32)
    return _build(rows_out, rows_in, nw, h, idx32)


# speedup vs baseline: 1.1265x; 1.0029x over previous
import functools

import jax
import jax.numpy as jnp
from jax import lax
from jax.experimental import pallas as pl
from jax.experimental.pallas import tpu as pltpu
from jax.experimental.pallas import tpu_sc as plsc

D = 256
CHUNK = 128
ZCH = 128
LANES = 16
MAXWIN = 26


@functools.partial(jax.jit, static_argnums=(0, 1, 2))
def _build(rows_out, rows_in, nw, h, idx32):
    per = (-(-rows_out // nw) + 7) // 8 * 8

    mesh = plsc.VectorSubcoreMesh(core_axis_name="c", subcore_axis_name="s")
    nc = mesh.num_cores

    @functools.partial(
        pl.kernel,
        out_type=jax.ShapeDtypeStruct((rows_out, D), jnp.float32),
        mesh=mesh,
        scratch_types=[
            pltpu.VMEM((ZCH, D), jnp.float32),
            pltpu.VMEM((2, CHUNK, D), jnp.float32),
            pltpu.VMEM((MAXWIN, CHUNK), jnp.int32),
            pltpu.VMEM((MAXWIN * CHUNK,), jnp.int32),
            pltpu.VMEM((LANES,), jnp.int32),
            pltpu.VMEM((LANES,), jnp.int32),
            pltpu.VMEM((LANES,), jnp.int32),
            pltpu.VMEM((LANES,), jnp.int32),
            pltpu.SemaphoreType.DMA,
            pltpu.SemaphoreType.DMA,
            pltpu.SemaphoreType.DMA,
            pltpu.SemaphoreType.DMA,
        ],
    )
    def unpool(h_hbm, idx_hbm, out_hbm, zeros_v, rows2_v, idx2_v,
               idxf_v, poss_v, pose_v, prbs_v, prbe_v, semz, semi, semh, sems):
        w = lax.axis_index("s") * nc + lax.axis_index("c")

        def zbody(i, carry):
            r = i // (D // LANES)
            c = (i % (D // LANES)) * LANES
            zeros_v[r, pl.ds(c, LANES)] = jnp.zeros((LANES,), jnp.float32)
            return carry

        lax.fori_loop(0, CHUNK * (D // LANES), zbody, 0)

        lo = w * per
        hi = jnp.minimum(lo + per, rows_out)
        nfull = (hi - lo) // ZCH

        iot = lax.iota(jnp.int32, LANES)
        ls = jnp.int32(0)
        hs = jnp.int32(rows_in)
        le = jnp.int32(0)
        he = jnp.int32(rows_in)
        for _ in range(4):
            st_s = (hs - ls + LANES - 1) // LANES
            st_e = (he - le + LANES - 1) // LANES
            poss_v[pl.ds(0, LANES)] = jnp.minimum(ls + iot * st_s, rows_in - 1)
            pose_v[pl.ds(0, LANES)] = jnp.minimum(le + iot * st_e, rows_in - 1)
            cp1 = pltpu.make_async_copy(idx_hbm.at[poss_v], prbs_v, semi)
            cp2 = pltpu.make_async_copy(idx_hbm.at[pose_v], prbe_v, semh)
            cp1.start()
            cp2.start()
            cp1.wait()
            cp2.wait()
            pvs = prbs_v[pl.ds(0, LANES)]
            pve = prbe_v[pl.ds(0, LANES)]
            cs = jnp.int32(0)
            ce = jnp.int32(0)
            for i in range(LANES):
                cs = cs + jnp.logical_and(
                    pvs[i] < lo, ls + i * st_s < hs
                ).astype(jnp.int32)
                ce = ce + jnp.logical_and(
                    pve[i] < hi, le + i * st_e < he
                ).astype(jnp.int32)
            new_ls = jnp.where(cs >= 1, ls + (cs - 1) * st_s + 1, ls)
            hs = jnp.minimum(ls + cs * st_s, hs)
            ls = new_ls
            new_le = jnp.where(ce >= 1, le + (ce - 1) * st_e + 1, le)
            he = jnp.minimum(le + ce * st_e, he)
            le = new_le
        s = ls
        e = le

        a0 = (s // 8) * 8
        nwin = (e - a0 + CHUNK - 1) // CHUNK

        def astart(j):
            return jnp.minimum(a0 + j * CHUNK, rows_in - CHUNK)

        def iissue(j, carry):
            pltpu.make_async_copy(
                idx_hbm.at[pl.ds(astart(j), CHUNK)], idx2_v.at[j], semi
            ).start()
            pltpu.make_async_copy(
                idx_hbm.at[pl.ds(astart(j), CHUNK)],
                idxf_v.at[pl.ds(j * CHUNK, CHUNK)], semi
            ).start()
            return carry

        lax.fori_loop(0, nwin, iissue, 0)

        @pl.when(nwin >= 1)
        def _():
            pltpu.make_async_copy(
                h_hbm.at[pl.ds(astart(0), CHUNK)], rows2_v.at[0], semh
            ).start()

        def idrain(j, carry):
            pltpu.make_async_copy(
                idx_hbm.at[pl.ds(0, CHUNK)], idx2_v.at[0], semi
            ).wait()
            return carry

        lax.fori_loop(0, 2 * nwin, idrain, 0)

        base = s - a0
        seglen = e - s

        def full_chunk(b):
            def bstep(i, c):
                blo, bhi = c
                mid = (blo + bhi) // 2
                v = idxf_v[pl.ds(base + mid, LANES)]
                lt = v[0] < b
                return (jnp.where(lt, mid + 1, blo), jnp.where(lt, bhi, mid))

            p, _ = lax.fori_loop(0, 12, bstep, (jnp.int32(0), seglen))
            v0 = idxf_v[pl.ds(base + p, LANES)]
            vn = idxf_v[pl.ds(base + p + ZCH - 1, LANES)]
            return jnp.logical_and(
                p + ZCH <= seglen,
                jnp.logical_and(v0[0] == b, vn[0] == b + ZCH - 1),
            )

        def zissue(j, nz):
            skip = full_chunk(lo + j * ZCH)

            @pl.when(jnp.logical_not(skip))
            def _():
                pltpu.make_async_copy(
                    zeros_v, out_hbm.at[pl.ds(lo + j * ZCH, ZCH)], semz
                ).start()

            return nz + 1 - skip.astype(jnp.int32)

        nz = lax.fori_loop(0, nfull, zissue, jnp.int32(0))
        skip_t = full_chunk(hi - ZCH)

        @pl.when(jnp.logical_not(skip_t))
        def _():
            pltpu.make_async_copy(
                zeros_v, out_hbm.at[pl.ds(hi - ZCH, ZCH)], semz
            ).start()

        nz = nz + 1 - skip_t.astype(jnp.int32)

        def zdrain(j, carry):
            pltpu.make_async_copy(
                zeros_v, out_hbm.at[pl.ds(lo, ZCH)], semz
            ).wait()
            return carry

        lax.fori_loop(0, nz, zdrain, 0)

        def scat(j, carry):
            b = j % 2
            pltpu.make_async_copy(
                h_hbm.at[pl.ds(0, CHUNK)], rows2_v.at[0], semh
            ).wait()

            @pl.when(j >= 1)
            def _():
                pltpu.make_async_copy(
                    rows2_v.at[0], out_hbm.at[idx2_v.at[0]], sems
                ).wait()

            @pl.when(j + 1 < nwin)
            def _():
                pltpu.make_async_copy(
                    h_hbm.at[pl.ds(astart(j + 1), CHUNK)], rows2_v.at[1 - b], semh
                ).start()

            pltpu.make_async_copy(
                rows2_v.at[b], out_hbm.at[idx2_v.at[j]], sems
            ).start()
            return carry

        lax.fori_loop(0, nwin, scat, 0)

        @pl.when(nwin >= 1)
        def _():
            pltpu.make_async_copy(
                rows2_v.at[0], out_hbm.at[idx2_v.at[0]], sems
            ).wait()

    return unpool(h, idx32)


def kernel(g, h, idx):
    rows_out = g.shape[0]
    rows_in = h.shape[0]
    info = plsc.get_sparse_core_info()
    nw = info.num_cores * info.num_subcores

    idx32 = idx.astype(jnp.int32)
    return _build(rows_out, rows_in, nw, h, idx32)
